# baseline (speedup 1.0000x reference)
import jax
import jax.numpy as jnp
from jax import lax
from jax.experimental import pallas as pl
from jax.experimental.pallas import tpu as pltpu
from jax.experimental.pallas import tpu_sc as plsc

N = 50000
E = 1600000
T = 12
IN = 16
H = 32

LANES = 128
CH = 14
NC, NS = 2, 16
NW = NC * NS
ER = E // LANES
ROWS_C0 = 336
ROWS_C1 = 448
ERP = NS * (ROWS_C0 + ROWS_C1)
PADE = ERP * LANES - E
DUMP_ROW = N + 8
NP = 51200
TILE_N = NP // NS
ZCH = 320
DEG_TILE = 3200
DEG_PAD = NS * DEG_TILE

NB2 = 512
NB = 400
GRID2 = (N + NB2 - 1) // NB2
GRID = N // NB

_MESH = plsc.VectorSubcoreMesh(core_axis_name="c", subcore_axis_name="s")
_SC_PARAMS = pltpu.CompilerParams(use_tc_tiling_on_sc=False)


def _deg_body(dst_hbm, out_hbm, didx, ones_v, buf, acc, semS):
    c = lax.axis_index("c")
    s = lax.axis_index("s")
    wid = c * NS + s

    for j in range(LANES // 16):
        ones_v[pl.ds(j * 16, 16)] = jnp.ones((16,), jnp.float32)

    @pl.loop(0, DEG_TILE, step=16)
    def _zb(i):
        buf[pl.ds(i, 16)] = jnp.zeros((16,), jnp.float32)

    pltpu.sync_copy(buf, acc.at[pl.ds(s * DEG_TILE, DEG_TILE)])
    plsc.subcore_barrier()

    start = jnp.where(c == 0, s * ROWS_C0, NS * ROWS_C0 + s * ROWS_C1)
    nch = jnp.where(c == 0, ROWS_C0 // CH, ROWS_C1 // CH)

    @pl.loop(0, nch)
    def _edges(i):
        pltpu.sync_copy(dst_hbm.at[pl.ds(start + i * CH, CH)], didx)

        @pl.loop(0, CH)
        def _fire(k):
            pltpu.async_copy(ones_v, acc.at[didx.at[k]], semS, add=True)

        @pl.loop(0, CH)
        def _drain(k):
            pltpu.make_async_copy(ones_v, acc.at[didx.at[k]], semS).wait()

    plsc.subcore_barrier()
    pltpu.sync_copy(acc.at[pl.ds(s * DEG_TILE, DEG_TILE)], buf)
    pltpu.sync_copy(buf, out_hbm.at[c].at[s])


_deg_call = pl.kernel(
    _deg_body,
    out_type=jax.ShapeDtypeStruct((NC, NS, DEG_TILE), jnp.float32),
    mesh=_MESH,
    compiler_params=_SC_PARAMS,
    scratch_types=[
        pltpu.VMEM((CH, LANES), jnp.int32),
        pltpu.VMEM((LANES,), jnp.float32),
        pltpu.VMEM((DEG_TILE,), jnp.float32),
        pltpu.VMEM_SHARED((DEG_PAD,), jnp.float32),
        pltpu.SemaphoreType.DMA,
    ],
)


def _agg_body(src_hbm, dst_hbm, y_hbm, out_hbm, sidxA, didxA, sidxtA, rowsA,
              sidxB, didxB, sidxtB, rowsB, zbuf, obuf, acc, semGA, semGB,
              semS):
    c = lax.axis_index("c")
    s = lax.axis_index("s")
    wid = c * NS + s

    @pl.loop(0, ZCH)
    def _zb(r):
        zbuf[r] = jnp.zeros((16,), jnp.float32)

    start = jnp.where(c == 0, s * ROWS_C0, NS * ROWS_C0 + s * ROWS_C1)
    npair = jnp.where(c == 0, ROWS_C0 // (2 * CH), ROWS_C1 // (2 * CH))

    @pl.loop(0, T)
    def _t(t):
        @pl.loop(0, TILE_N // ZCH)
        def _z(k):
            pltpu.sync_copy(zbuf, acc.at[pl.ds(s * TILE_N + k * ZCH, ZCH)])

        plsc.subcore_barrier()

        def load_fire(i, sidx, didx, sidxt, rows, semG):
            r0 = start + i * CH
            pltpu.sync_copy(src_hbm.at[pl.ds(r0, CH)], sidx)
            pltpu.sync_copy(dst_hbm.at[pl.ds(r0, CH)], didx)

            @pl.loop(0, CH)
            def _idx(k):
                for j in range(LANES // 16):
                    sl = pl.ds(j * 16, 16)
                    sidxt[k, sl] = sidx[k, sl] * T + t

            @pl.loop(0, CH)
            def _fire(k):
                pltpu.async_copy(y_hbm.at[sidxt.at[k]], rows.at[k], semG)

        def wait_scatter(sidxt, didx, rows, semG):
            @pl.loop(0, CH)
            def _wg(k):
                pltpu.make_async_copy(y_hbm.at[sidxt.at[k]], rows.at[k],
                                      semG).wait()

            @pl.loop(0, CH)
            def _fs(k):
                pltpu.async_copy(rows.at[k], acc.at[didx.at[k]], semS,
                                 add=True)

        def drain_scatter(didx, rows):
            @pl.loop(0, CH)
            def _ds(k):
                pltpu.make_async_copy(rows.at[k], acc.at[didx.at[k]],
                                      semS).wait()

        @pl.loop(0, npair)
        def _pair(j):
            load_fire(2 * j, sidxA, didxA, sidxtA, rowsA, semGA)
            load_fire(2 * j + 1, sidxB, didxB, sidxtB, rowsB, semGB)
            wait_scatter(sidxtA, didxA, rowsA, semGA)
            wait_scatter(sidxtB, didxB, rowsB, semGB)
            drain_scatter(didxA, rowsA)
            drain_scatter(didxB, rowsB)

        plsc.subcore_barrier()

        @pl.loop(0, TILE_N // ZCH)
        def _o(k):
            base = s * TILE_N + k * ZCH
            pltpu.sync_copy(acc.at[pl.ds(base, ZCH)],
                            out_hbm.at[t].at[c].at[pl.ds(base, ZCH)])


_agg_call = pl.kernel(
    _agg_body,
    out_type=jax.ShapeDtypeStruct((T, NC, NP, IN), jnp.float32),
    mesh=_MESH,
    compiler_params=_SC_PARAMS,
    scratch_types=[
        pltpu.VMEM((CH, LANES), jnp.int32),
        pltpu.VMEM((CH, LANES), jnp.int32),
        pltpu.VMEM((CH, LANES), jnp.int32),
        pltpu.VMEM((CH, LANES, IN), jnp.float32),
        pltpu.VMEM((CH, LANES), jnp.int32),
        pltpu.VMEM((CH, LANES), jnp.int32),
        pltpu.VMEM((CH, LANES), jnp.int32),
        pltpu.VMEM((CH, LANES, IN), jnp.float32),
        pltpu.VMEM((ZCH, IN), jnp.float32),
        pltpu.VMEM((ZCH, IN), jnp.float32),
        pltpu.VMEM_SHARED((NP, IN), jnp.float32),
        pltpu.SemaphoreType.DMA,
        pltpu.SemaphoreType.DMA,
        pltpu.SemaphoreType.DMA,
    ],
)


def _prep_body(degp_ref, x_ref, dinv_ref, y_ref):
    deg = degp_ref[:, 0] + degp_ref[:, 1] + 1.0
    dinv = lax.rsqrt(deg)[:, None]
    dinv_ref[...] = dinv
    y_ref[...] = x_ref[...] * dinv[:, None]


_prep_call = pl.pallas_call(
    _prep_body,
    grid=(GRID2,),
    in_specs=[
        pl.BlockSpec((NB2, 2), lambda i: (i, 0)),
        pl.BlockSpec((NB2, T, IN), lambda i: (i, 0, 0)),
    ],
    out_specs=[
        pl.BlockSpec((NB2, 1), lambda i: (i, 0)),
        pl.BlockSpec((NB2, T, IN), lambda i: (i, 0, 0)),
    ],
    out_shape=[
        jax.ShapeDtypeStruct((N, 1), jnp.float32),
        jax.ShapeDtypeStruct((N, T, IN), jnp.float32),
    ],
)


def _gru_body(agg_ref, y_ref, dinv_ref, W1_ref, b1_ref, Wih_ref, bih_ref,
              Whh_ref, bhh_ref, Wfc_ref, bfc_ref, out_ref):
    dinv = dinv_ref[...]
    W1 = W1_ref[...]
    b1 = b1_ref[...]
    Wih = Wih_ref[...]
    bih = bih_ref[...]
    Whh = Whh_ref[...]
    bhh = bhh_ref[...]

    h = jnp.zeros((NB, H), jnp.float32)
    for t in range(T):
        z = dinv * (agg_ref[t, 0] + agg_ref[t, 1] + y_ref[:, t, :])
        u = jnp.maximum(
            jnp.dot(z, W1, preferred_element_type=jnp.float32) + b1, 0.0)
        gi = jnp.dot(u, Wih, preferred_element_type=jnp.float32) + bih
        gh = jnp.dot(h, Whh, preferred_element_type=jnp.float32) + bhh
        r = jax.nn.sigmoid(gi[:, :H] + gh[:, :H])
        zz = jax.nn.sigmoid(gi[:, H:2 * H] + gh[:, H:2 * H])
        n = jnp.tanh(gi[:, 2 * H:] + r * gh[:, 2 * H:])
        h = (1.0 - zz) * n + zz * h

    out_ref[...] = (
        jnp.dot(h, Wfc_ref[...], preferred_element_type=jnp.float32)
        + bfc_ref[...])


_gru_call = pl.pallas_call(
    _gru_body,
    grid=(GRID,),
    in_specs=[
        pl.BlockSpec((T, NC, NB, IN), lambda i: (0, 0, i, 0)),
        pl.BlockSpec((NB, T, IN), lambda i: (i, 0, 0)),
        pl.BlockSpec((NB, 1), lambda i: (i, 0)),
        pl.BlockSpec((IN, H), lambda i: (0, 0)),
        pl.BlockSpec((1, H), lambda i: (0, 0)),
        pl.BlockSpec((H, 3 * H), lambda i: (0, 0)),
        pl.BlockSpec((1, 3 * H), lambda i: (0, 0)),
        pl.BlockSpec((H, 3 * H), lambda i: (0, 0)),
        pl.BlockSpec((1, 3 * H), lambda i: (0, 0)),
        pl.BlockSpec((H, 1), lambda i: (0, 0)),
        pl.BlockSpec((1, 1), lambda i: (0, 0)),
    ],
    out_specs=pl.BlockSpec((NB, 1), lambda i: (i, 0)),
    out_shape=jax.ShapeDtypeStruct((N, 1), jnp.float32),
)


def kernel(x, edge_index, W1, b1, W_ih, W_hh, b_ih, b_hh, W_fc, b_fc):
    src_pad = jnp.concatenate(
        [edge_index[0], jnp.zeros((PADE,), jnp.int32)])
    dst_pad = jnp.concatenate(
        [edge_index[1], jnp.full((PADE,), DUMP_ROW, jnp.int32)])
    src2d = src_pad.reshape(ERP, LANES)
    dst2d = dst_pad.reshape(ERP, LANES)

    deg_parts = _deg_call(dst2d)
    degp = deg_parts.reshape(NC, DEG_PAD)[:, :N].T

    dinv, y = _prep_call(degp, x)
    agg = _agg_call(src2d, dst2d, y.reshape(N * T, IN))

    pred = _gru_call(
        agg, y, dinv,
        W1, b1.reshape(1, H),
        W_ih.T, b_ih.reshape(1, 3 * H),
        W_hh.T, b_hh.reshape(1, 3 * H),
        W_fc, b_fc.reshape(1, 1),
    )
    return pred.reshape(N)


# speedup vs baseline: 1.0801x; 1.0801x over previous
import jax
import jax.numpy as jnp
from jax import lax
from jax.experimental import pallas as pl
from jax.experimental.pallas import tpu as pltpu
from jax.experimental.pallas import tpu_sc as plsc

N = 50000
E = 1600000
T = 12
IN = 16
H = 32

LANES = 128
CH = 14
NC, NS = 2, 16
NW = NC * NS
ER = E // LANES
ROWS_C0 = 448
ROWS_C1 = 336
ERP = NS * (ROWS_C0 + ROWS_C1)
PADE = ERP * LANES - E
DUMP_ROW = N + 8
NP = 51200
TILE_N = NP // NS
ZCH = 320
DEG_TILE = 3200
DEG_PAD = NS * DEG_TILE

NB2 = 512
NB = 400
GRID2 = (N + NB2 - 1) // NB2
GRID = N // NB

_MESH = plsc.VectorSubcoreMesh(core_axis_name="c", subcore_axis_name="s")
_SC_PARAMS = pltpu.CompilerParams(use_tc_tiling_on_sc=False)


def _deg_body(dst_hbm, out_hbm, didx, ones_v, buf, acc, semS):
    c = lax.axis_index("c")
    s = lax.axis_index("s")
    wid = c * NS + s

    for j in range(LANES // 16):
        ones_v[pl.ds(j * 16, 16)] = jnp.ones((16,), jnp.float32)

    @pl.loop(0, DEG_TILE, step=16)
    def _zb(i):
        buf[pl.ds(i, 16)] = jnp.zeros((16,), jnp.float32)

    pltpu.sync_copy(buf, acc.at[pl.ds(s * DEG_TILE, DEG_TILE)])
    plsc.subcore_barrier()

    start = jnp.where(c == 0, s * ROWS_C0, NS * ROWS_C0 + s * ROWS_C1)
    nch = jnp.where(c == 0, ROWS_C0 // CH, ROWS_C1 // CH)

    @pl.loop(0, nch)
    def _edges(i):
        pltpu.sync_copy(dst_hbm.at[pl.ds(start + i * CH, CH)], didx)

        @pl.loop(0, CH)
        def _fire(k):
            pltpu.async_copy(ones_v, acc.at[didx.at[k]], semS, add=True)

        @pl.loop(0, CH)
        def _drain(k):
            pltpu.make_async_copy(ones_v, acc.at[didx.at[k]], semS).wait()

    plsc.subcore_barrier()
    pltpu.sync_copy(acc.at[pl.ds(s * DEG_TILE, DEG_TILE)], buf)
    pltpu.sync_copy(buf, out_hbm.at[c].at[s])


_deg_call = pl.kernel(
    _deg_body,
    out_type=jax.ShapeDtypeStruct((NC, NS, DEG_TILE), jnp.float32),
    mesh=_MESH,
    compiler_params=_SC_PARAMS,
    scratch_types=[
        pltpu.VMEM((CH, LANES), jnp.int32),
        pltpu.VMEM((LANES,), jnp.float32),
        pltpu.VMEM((DEG_TILE,), jnp.float32),
        pltpu.VMEM_SHARED((DEG_PAD,), jnp.float32),
        pltpu.SemaphoreType.DMA,
    ],
)


def _agg_body(src_hbm, dst_hbm, y_hbm, out_hbm, sidxA, didxA, sidxtA, rowsA,
              sidxB, didxB, sidxtB, rowsB, zbuf, obuf, acc, semGA, semGB,
              semS):
    c = lax.axis_index("c")
    s = lax.axis_index("s")
    wid = c * NS + s

    @pl.loop(0, ZCH)
    def _zb(r):
        zbuf[r] = jnp.zeros((16,), jnp.float32)

    start = jnp.where(c == 0, s * ROWS_C0, NS * ROWS_C0 + s * ROWS_C1)
    npair = jnp.where(c == 0, ROWS_C0 // (2 * CH), ROWS_C1 // (2 * CH))

    @pl.loop(0, T)
    def _t(t):
        @pl.loop(0, TILE_N // ZCH)
        def _z(k):
            pltpu.sync_copy(zbuf, acc.at[pl.ds(s * TILE_N + k * ZCH, ZCH)])

        plsc.subcore_barrier()

        def load_fire(i, sidx, didx, sidxt, rows, semG):
            r0 = start + i * CH
            pltpu.sync_copy(src_hbm.at[pl.ds(r0, CH)], sidx)
            pltpu.sync_copy(dst_hbm.at[pl.ds(r0, CH)], didx)

            @pl.loop(0, CH)
            def _idx(k):
                for j in range(LANES // 16):
                    sl = pl.ds(j * 16, 16)
                    sidxt[k, sl] = sidx[k, sl] * T + t

            @pl.loop(0, CH)
            def _fire(k):
                pltpu.async_copy(y_hbm.at[sidxt.at[k]], rows.at[k], semG)

        def wait_scatter(sidxt, didx, rows, semG):
            @pl.loop(0, CH)
            def _wg(k):
                pltpu.make_async_copy(y_hbm.at[sidxt.at[k]], rows.at[k],
                                      semG).wait()

            @pl.loop(0, CH)
            def _fs(k):
                pltpu.async_copy(rows.at[k], acc.at[didx.at[k]], semS,
                                 add=True)

        def drain_scatter(didx, rows):
            @pl.loop(0, CH)
            def _ds(k):
                pltpu.make_async_copy(rows.at[k], acc.at[didx.at[k]],
                                      semS).wait()

        @pl.loop(0, npair)
        def _pair(j):
            load_fire(2 * j, sidxA, didxA, sidxtA, rowsA, semGA)
            load_fire(2 * j + 1, sidxB, didxB, sidxtB, rowsB, semGB)
            wait_scatter(sidxtA, didxA, rowsA, semGA)
            wait_scatter(sidxtB, didxB, rowsB, semGB)
            drain_scatter(didxA, rowsA)
            drain_scatter(didxB, rowsB)

        plsc.subcore_barrier()

        @pl.loop(0, TILE_N // ZCH)
        def _o(k):
            base = s * TILE_N + k * ZCH
            pltpu.sync_copy(acc.at[pl.ds(base, ZCH)],
                            out_hbm.at[t].at[c].at[pl.ds(base, ZCH)])


_agg_call = pl.kernel(
    _agg_body,
    out_type=jax.ShapeDtypeStruct((T, NC, NP, IN), jnp.float32),
    mesh=_MESH,
    compiler_params=_SC_PARAMS,
    scratch_types=[
        pltpu.VMEM((CH, LANES), jnp.int32),
        pltpu.VMEM((CH, LANES), jnp.int32),
        pltpu.VMEM((CH, LANES), jnp.int32),
        pltpu.VMEM((CH, LANES, IN), jnp.float32),
        pltpu.VMEM((CH, LANES), jnp.int32),
        pltpu.VMEM((CH, LANES), jnp.int32),
        pltpu.VMEM((CH, LANES), jnp.int32),
        pltpu.VMEM((CH, LANES, IN), jnp.float32),
        pltpu.VMEM((ZCH, IN), jnp.float32),
        pltpu.VMEM((ZCH, IN), jnp.float32),
        pltpu.VMEM_SHARED((NP, IN), jnp.float32),
        pltpu.SemaphoreType.DMA,
        pltpu.SemaphoreType.DMA,
        pltpu.SemaphoreType.DMA,
    ],
)


def _prep_body(degp_ref, x_ref, dinv_ref, y_ref):
    deg = degp_ref[:, 0] + degp_ref[:, 1] + 1.0
    dinv = lax.rsqrt(deg)[:, None]
    dinv_ref[...] = dinv
    y_ref[...] = x_ref[...] * dinv[:, None]


_prep_call = pl.pallas_call(
    _prep_body,
    grid=(GRID2,),
    in_specs=[
        pl.BlockSpec((NB2, 2), lambda i: (i, 0)),
        pl.BlockSpec((NB2, T, IN), lambda i: (i, 0, 0)),
    ],
    out_specs=[
        pl.BlockSpec((NB2, 1), lambda i: (i, 0)),
        pl.BlockSpec((NB2, T, IN), lambda i: (i, 0, 0)),
    ],
    out_shape=[
        jax.ShapeDtypeStruct((N, 1), jnp.float32),
        jax.ShapeDtypeStruct((N, T, IN), jnp.float32),
    ],
)


def _gru_body(agg_ref, y_ref, dinv_ref, W1_ref, b1_ref, Wih_ref, bih_ref,
              Whh_ref, bhh_ref, Wfc_ref, bfc_ref, out_ref):
    dinv = dinv_ref[...]
    W1 = W1_ref[...]
    b1 = b1_ref[...]
    Wih = Wih_ref[...]
    bih = bih_ref[...]
    Whh = Whh_ref[...]
    bhh = bhh_ref[...]

    h = jnp.zeros((NB, H), jnp.float32)
    for t in range(T):
        z = dinv * (agg_ref[t, 0] + agg_ref[t, 1] + y_ref[:, t, :])
        u = jnp.maximum(
            jnp.dot(z, W1, preferred_element_type=jnp.float32) + b1, 0.0)
        gi = jnp.dot(u, Wih, preferred_element_type=jnp.float32) + bih
        gh = jnp.dot(h, Whh, preferred_element_type=jnp.float32) + bhh
        r = jax.nn.sigmoid(gi[:, :H] + gh[:, :H])
        zz = jax.nn.sigmoid(gi[:, H:2 * H] + gh[:, H:2 * H])
        n = jnp.tanh(gi[:, 2 * H:] + r * gh[:, 2 * H:])
        h = (1.0 - zz) * n + zz * h

    out_ref[...] = (
        jnp.dot(h, Wfc_ref[...], preferred_element_type=jnp.float32)
        + bfc_ref[...])


_gru_call = pl.pallas_call(
    _gru_body,
    grid=(GRID,),
    in_specs=[
        pl.BlockSpec((T, NC, NB, IN), lambda i: (0, 0, i, 0)),
        pl.BlockSpec((NB, T, IN), lambda i: (i, 0, 0)),
        pl.BlockSpec((NB, 1), lambda i: (i, 0)),
        pl.BlockSpec((IN, H), lambda i: (0, 0)),
        pl.BlockSpec((1, H), lambda i: (0, 0)),
        pl.BlockSpec((H, 3 * H), lambda i: (0, 0)),
        pl.BlockSpec((1, 3 * H), lambda i: (0, 0)),
        pl.BlockSpec((H, 3 * H), lambda i: (0, 0)),
        pl.BlockSpec((1, 3 * H), lambda i: (0, 0)),
        pl.BlockSpec((H, 1), lambda i: (0, 0)),
        pl.BlockSpec((1, 1), lambda i: (0, 0)),
    ],
    out_specs=pl.BlockSpec((NB, 1), lambda i: (i, 0)),
    out_shape=jax.ShapeDtypeStruct((N, 1), jnp.float32),
)


def kernel(x, edge_index, W1, b1, W_ih, W_hh, b_ih, b_hh, W_fc, b_fc):
    src_pad = jnp.concatenate(
        [edge_index[0], jnp.zeros((PADE,), jnp.int32)])
    dst_pad = jnp.concatenate(
        [edge_index[1], jnp.full((PADE,), DUMP_ROW, jnp.int32)])
    src2d = src_pad.reshape(ERP, LANES)
    dst2d = dst_pad.reshape(ERP, LANES)

    deg_parts = _deg_call(dst2d)
    degp = deg_parts.reshape(NC, DEG_PAD)[:, :N].T

    dinv, y = _prep_call(degp, x)
    agg = _agg_call(src2d, dst2d, y.reshape(N * T, IN))

    pred = _gru_call(
        agg, y, dinv,
        W1, b1.reshape(1, H),
        W_ih.T, b_ih.reshape(1, 3 * H),
        W_hh.T, b_hh.reshape(1, 3 * H),
        W_fc, b_fc.reshape(1, 1),
    )
    return pred.reshape(N)
